# baseline (speedup 1.0000x reference)
import functools

import jax
import jax.numpy as jnp
from jax import lax
from jax.experimental import pallas as pl
from jax.experimental.pallas import tpu as pltpu
from jax.experimental.pallas import tpu_sc as plsc

_D = 32


@functools.lru_cache(maxsize=None)
def _make_gather(b_dim: int, s_dim: int, vocab: int):
    info = plsc.get_sparse_core_info()
    nc, ns = info.num_cores, info.num_subcores
    nw = nc * ns
    assert b_dim % nw == 0
    d0_per_w = b_dim // nw
    lo_n = min(16, s_dim)

    mesh = plsc.VectorSubcoreMesh(core_axis_name="c", subcore_axis_name="s")

    @functools.partial(
        pl.kernel,
        mesh=mesh,
        out_type=jax.ShapeDtypeStruct((b_dim, s_dim, _D), jnp.float32),
        scratch_types=[
            pltpu.VMEM((d0_per_w, s_dim), jnp.int32),
            [pltpu.VMEM((8, _D), jnp.float32) for _ in range(s_dim)],
            [pltpu.VMEM((s_dim, _D), jnp.float32) for _ in range(2)],
            [pltpu.SemaphoreType.DMA for _ in range(s_dim)],
            [pltpu.SemaphoreType.DMA for _ in range(2)],
        ],
    )
    def gather_kernel(table_hbm, idx_hbm, out_hbm, idx_v, blk, stg, bsem, osem):
        wid = lax.axis_index("s") * nc + lax.axis_index("c")
        d0_base = pl.multiple_of(wid * d0_per_w, 8)
        pltpu.sync_copy(idx_hbm.at[pl.ds(d0_base, d0_per_w)], idx_v)

        def row_vecs(r):
            w0 = idx_v[r, pl.ds(0, 16)]
            w1 = idx_v[r, pl.ds(s_dim - 16, 16)]
            return w0, w1

        def lane(vecs, s):
            w0, w1 = vecs
            return w0[s] if s < lo_n else w1[s - (s_dim - 16)]

        def issue(e, s):
            b0 = pl.multiple_of((e >> 3) * 8, 8)
            pltpu.async_copy(table_hbm.at[pl.ds(b0, 8)], blk[s], bsem[s])

        def wait_blk(s):
            pltpu.make_async_copy(table_hbm.at[pl.ds(0, 8)], blk[s], bsem[s]).wait()

        def wait_out(dd):
            pltpu.make_async_copy(stg[dd], out_hbm.at[d0_base], osem[dd]).wait()

        v0 = row_vecs(0)
        for s in range(s_dim):
            issue(lane(v0, s), s)

        def body(sup, _):
            for dd in range(2):
                r = sup * 2 + dd
                rn = jnp.minimum(r + 1, d0_per_w - 1)
                cur = row_vecs(r)
                nxt = row_vecs(rn)

                @pl.when(sup > 0)
                def _():
                    wait_out(dd)

                for s in range(s_dim):
                    wait_blk(s)
                    sub = lane(cur, s) & 7
                    stg[dd][s, pl.ds(0, 16)] = blk[s][sub, pl.ds(0, 16)]
                    stg[dd][s, pl.ds(16, 16)] = blk[s][sub, pl.ds(16, 16)]
                    en = lane(nxt, s)

                    @pl.when(r + 1 < d0_per_w)
                    def _():
                        issue(en, s)

                pltpu.async_copy(stg[dd], out_hbm.at[d0_base + r], osem[dd])
            return 0

        lax.fori_loop(0, d0_per_w // 2, body, 0)
        for dd in range(2):
            wait_out(dd)

    return gather_kernel


def kernel(x, table):
    b, s = x.shape
    return _make_gather(b, s, table.shape[0])(table, x.astype(jnp.int32))


# speedup vs baseline: 2.2973x; 2.2973x over previous
import functools

import jax
import jax.numpy as jnp
from jax import lax
from jax.experimental import pallas as pl
from jax.experimental.pallas import tpu as pltpu
from jax.experimental.pallas import tpu_sc as plsc

_D = 32


@functools.lru_cache(maxsize=None)
def _make_detile(vocab: int):
    info = plsc.get_sparse_core_info()
    nc, ns = info.num_cores, info.num_subcores
    nw = nc * ns
    blocks = vocab // 8
    per_w, extra = divmod(blocks, nw)
    chunk = 248
    n_chunks = (per_w * 8) // chunk
    assert n_chunks * chunk == per_w * 8

    mesh = plsc.VectorSubcoreMesh(core_axis_name="c", subcore_axis_name="s")

    @functools.partial(
        pl.kernel,
        mesh=mesh,
        out_type=jax.ShapeDtypeStruct((vocab * _D,), jnp.float32),
        scratch_types=[
            [pltpu.VMEM((chunk, _D), jnp.float32) for _ in range(2)],
            [pltpu.VMEM((chunk * _D,), jnp.float32) for _ in range(2)],
            [pltpu.SemaphoreType.DMA for _ in range(2)],
            [pltpu.SemaphoreType.DMA for _ in range(2)],
            pltpu.VMEM((8, _D), jnp.float32),
            pltpu.VMEM((8 * _D,), jnp.float32),
            pltpu.SemaphoreType.DMA,
        ],
    )
    def detile_kernel(table_hbm, out_hbm, av, bv, isem, osem, a8, b8, sem8):
        wid = lax.axis_index("s") * nc + lax.axis_index("c")
        base = pl.multiple_of(
            (wid * per_w + jnp.minimum(wid, extra)) * 8, 8
        )

        def in_dma(i, b):
            r0 = pl.multiple_of(base + i * chunk, 8)
            return pltpu.make_async_copy(table_hbm.at[pl.ds(r0, chunk)], av[b], isem[b])

        def out_dma(i, b):
            o0 = pl.multiple_of((base + i * chunk) * _D, 8)
            return pltpu.make_async_copy(bv[b], out_hbm.at[pl.ds(o0, chunk * _D)], osem[b])

        def relayout(a, bvec, rows):
            def grp(g, _):
                for j in range(8):
                    rr = g * 8 + j
                    bvec[pl.ds(rr * _D, 16)] = a[rr, pl.ds(0, 16)]
                    bvec[pl.ds(rr * _D + 16, 16)] = a[rr, pl.ds(16, 16)]
                return 0

            lax.fori_loop(0, rows // 8, grp, 0)

        in_dma(0, 0).start()
        in_dma(1, 1).start()

        def body(sup, _):
            for b in range(2):
                i = sup * 2 + b
                in_dma(i, b).wait()
                @pl.when(sup > 0)
                def _():
                    out_dma(i - 2, b).wait()

                relayout(av[b], bv[b], chunk)
                out_dma(i, b).start()

                @pl.when(i + 2 < n_chunks)
                def _():
                    in_dma(i + 2, b).start()

            return 0

        lax.fori_loop(0, n_chunks // 2, body, 0)
        for i in range(2):
            out_dma(n_chunks - 2 + i, (n_chunks - 2 + i) % 2).wait()

        @pl.when(wid < extra)
        def _():
            r0 = pl.multiple_of(base + per_w * 8, 8)
            pltpu.sync_copy(table_hbm.at[pl.ds(r0, 8)], a8)
            for j in range(8):
                b8[pl.ds(j * _D, 16)] = a8[j, pl.ds(0, 16)]
                b8[pl.ds(j * _D + 16, 16)] = a8[j, pl.ds(16, 16)]
            pltpu.sync_copy(b8, out_hbm.at[pl.ds(r0 * _D, 8 * _D)])

    return detile_kernel


@functools.lru_cache(maxsize=None)
def _make_gather(n_rows: int, vocab: int):
    info = plsc.get_sparse_core_info()
    nc, ns = info.num_cores, info.num_subcores
    nw = nc * ns
    assert n_rows % nw == 0
    b_per_w = n_rows // nw
    chunk = 832
    while b_per_w % chunk:
        chunk //= 2
    n_chunks = b_per_w // chunk
    nbuf = min(2, n_chunks)

    mesh = plsc.VectorSubcoreMesh(core_axis_name="c", subcore_axis_name="s")

    @functools.partial(
        pl.kernel,
        mesh=mesh,
        out_type=jax.ShapeDtypeStruct((n_rows, _D), jnp.float32),
        scratch_types=[
            pltpu.VMEM((b_per_w,), jnp.int32),
            [pltpu.VMEM((chunk, _D), jnp.float32) for _ in range(nbuf)],
            [pltpu.SemaphoreType.DMA for _ in range(nbuf)],
            [pltpu.SemaphoreType.DMA for _ in range(nbuf)],
        ],
        compiler_params=pltpu.CompilerParams(use_tc_tiling_on_sc=False),
    )
    def gather_kernel(table_hbm, idx_hbm, out_hbm, idx_v, rows, gsem, osem):
        wid = lax.axis_index("s") * nc + lax.axis_index("c")
        base = wid * b_per_w
        pltpu.sync_copy(idx_hbm.at[pl.ds(base, b_per_w)], idx_v)

        def gather(i, b):
            return pltpu.make_async_copy(
                table_hbm.at[idx_v.at[pl.ds(i * chunk, chunk)]], rows[b], gsem[b]
            )

        def out_copy(i, b):
            return pltpu.make_async_copy(
                rows[b], out_hbm.at[pl.ds(base + i * chunk, chunk)], osem[b]
            )

        for i in range(nbuf):
            gather(i, i).start()
        for i in range(n_chunks):
            b = i % nbuf
            gather(i, b).wait()
            out_copy(i, b).start()
            j = i + nbuf
            if j < n_chunks:
                out_copy(i, b).wait()
                gather(j, b).start()
        for i in range(n_chunks - nbuf, n_chunks):
            out_copy(i, i % nbuf).wait()

    return gather_kernel


def kernel(x, table):
    b, s = x.shape
    vocab = table.shape[0]
    idx_flat = x.reshape(b * s).astype(jnp.int32)
    flat = _make_detile(vocab)(table)
    table_lin = flat.reshape(vocab, _D)
    out = _make_gather(b * s, vocab)(table_lin, idx_flat)
    return out.reshape(b, s, _D)


# speedup vs baseline: 2.3068x; 1.0041x over previous
import functools

import jax
import jax.numpy as jnp
from jax import lax
from jax.experimental import pallas as pl
from jax.experimental.pallas import tpu as pltpu
from jax.experimental.pallas import tpu_sc as plsc

_D = 32


@functools.lru_cache(maxsize=None)
def _make_gather(n_rows: int, vocab: int):
    info = plsc.get_sparse_core_info()
    nc, ns = info.num_cores, info.num_subcores
    nw = nc * ns
    assert n_rows % nw == 0
    b_per_w = n_rows // nw
    chunk = 832
    while b_per_w % chunk:
        chunk //= 2
    n_chunks = b_per_w // chunk
    nbuf = min(2, n_chunks)

    mesh = plsc.VectorSubcoreMesh(core_axis_name="c", subcore_axis_name="s")

    @functools.partial(
        pl.kernel,
        mesh=mesh,
        out_type=jax.ShapeDtypeStruct((n_rows, _D), jnp.float32),
        scratch_types=[
            pltpu.VMEM((b_per_w,), jnp.int32),
            [pltpu.VMEM((chunk, _D), jnp.float32) for _ in range(nbuf)],
            [pltpu.SemaphoreType.DMA for _ in range(nbuf)],
            [pltpu.SemaphoreType.DMA for _ in range(nbuf)],
        ],
        compiler_params=pltpu.CompilerParams(use_tc_tiling_on_sc=False),
    )
    def gather_kernel(table_hbm, idx_hbm, out_hbm, idx_v, rows, gsem, osem):
        wid = lax.axis_index("s") * nc + lax.axis_index("c")
        base = wid * b_per_w
        pltpu.sync_copy(idx_hbm.at[pl.ds(base, b_per_w)], idx_v)

        def gather(i, b):
            return pltpu.make_async_copy(
                table_hbm.at[idx_v.at[pl.ds(i * chunk, chunk)]], rows[b], gsem[b]
            )

        def out_copy(i, b):
            return pltpu.make_async_copy(
                rows[b], out_hbm.at[pl.ds(base + i * chunk, chunk)], osem[b]
            )

        for i in range(nbuf):
            gather(i, i).start()
        for i in range(n_chunks):
            b = i % nbuf
            gather(i, b).wait()
            out_copy(i, b).start()
            j = i + nbuf
            if j < n_chunks:
                out_copy(i, b).wait()
                gather(j, b).start()
        for i in range(n_chunks - nbuf, n_chunks):
            out_copy(i, i % nbuf).wait()

    return gather_kernel


@functools.lru_cache(maxsize=None)
def _make_retile(b_dim: int, s_dim: int):
    info = plsc.get_sparse_core_info()
    nc, ns = info.num_cores, info.num_subcores
    nw = nc * ns
    assert b_dim % nw == 0
    d0_per_w = b_dim // nw
    row_elems = s_dim * _D
    grp = 4
    n_grp = d0_per_w // grp
    assert n_grp * grp == d0_per_w

    mesh = plsc.VectorSubcoreMesh(core_axis_name="c", subcore_axis_name="s")

    @functools.partial(
        pl.kernel,
        mesh=mesh,
        out_type=jax.ShapeDtypeStruct((b_dim, s_dim, _D), jnp.float32),
        scratch_types=[
            [pltpu.VMEM((grp * row_elems,), jnp.float32) for _ in range(2)],
            [pltpu.VMEM((grp, s_dim, _D), jnp.float32) for _ in range(2)],
            [pltpu.SemaphoreType.DMA for _ in range(2)],
            [pltpu.SemaphoreType.DMA for _ in range(2)],
        ],
    )
    def retile_kernel(flat_hbm, out_hbm, av, bv, isem, osem):
        wid = lax.axis_index("s") * nc + lax.axis_index("c")
        d0_base = pl.multiple_of(wid * d0_per_w, 8)

        def in_dma(g, b):
            o0 = pl.multiple_of((d0_base + g * grp) * row_elems, 8)
            return pltpu.make_async_copy(
                flat_hbm.at[pl.ds(o0, grp * row_elems)], av[b], isem[b]
            )

        def out_dma(g, b):
            r0 = pl.multiple_of(d0_base + g * grp, 4)
            return pltpu.make_async_copy(
                bv[b], out_hbm.at[pl.ds(r0, grp)], osem[b]
            )

        def relayout(a, b3):
            for d in range(grp):
                for s in range(s_dim):
                    off = (d * s_dim + s) * _D
                    b3[d, s, pl.ds(0, 16)] = a[pl.ds(off, 16)]
                    b3[d, s, pl.ds(16, 16)] = a[pl.ds(off + 16, 16)]

        in_dma(0, 0).start()
        in_dma(1, 1).start()

        def body(sup, _):
            for b in range(2):
                g = sup * 2 + b
                in_dma(g, b).wait()

                @pl.when(sup > 0)
                def _():
                    out_dma(g - 2, b).wait()

                relayout(av[b], bv[b])
                out_dma(g, b).start()

                @pl.when(g + 2 < n_grp)
                def _():
                    in_dma(g + 2, b).start()

            return 0

        lax.fori_loop(0, n_grp // 2, body, 0)
        for i in range(2):
            out_dma(n_grp - 2 + i, (n_grp - 2 + i) % 2).wait()

    return retile_kernel


def kernel(x, table):
    b, s = x.shape
    vocab = table.shape[0]
    idx_flat = x.reshape(b * s).astype(jnp.int32)
    flat = _make_gather(b * s, vocab)(table, idx_flat)
    return _make_retile(b, s)(flat.reshape(b * s * _D))


# speedup vs baseline: 2.4402x; 1.0578x over previous
import functools

import jax
import jax.numpy as jnp
from jax import lax
from jax.experimental import pallas as pl
from jax.experimental.pallas import tpu as pltpu
from jax.experimental.pallas import tpu_sc as plsc

_D = 32


@functools.lru_cache(maxsize=None)
def _make_gather(n_rows: int, vocab: int):
    info = plsc.get_sparse_core_info()
    nc, ns = info.num_cores, info.num_subcores
    nw = nc * ns
    assert n_rows % nw == 0
    b_per_w = n_rows // nw
    chunk = 832
    while b_per_w % chunk:
        chunk //= 2
    n_chunks = b_per_w // chunk
    nbuf = min(4, n_chunks)

    mesh = plsc.VectorSubcoreMesh(core_axis_name="c", subcore_axis_name="s")

    @functools.partial(
        pl.kernel,
        mesh=mesh,
        out_type=jax.ShapeDtypeStruct((n_rows, _D), jnp.float32),
        scratch_types=[
            pltpu.VMEM((b_per_w,), jnp.int32),
            [pltpu.VMEM((chunk, _D), jnp.float32) for _ in range(nbuf)],
            [pltpu.SemaphoreType.DMA for _ in range(nbuf)],
            [pltpu.SemaphoreType.DMA for _ in range(nbuf)],
        ],
        compiler_params=pltpu.CompilerParams(use_tc_tiling_on_sc=False),
    )
    def gather_kernel(table_hbm, idx_hbm, out_hbm, idx_v, rows, gsem, osem):
        wid = lax.axis_index("s") * nc + lax.axis_index("c")
        base = wid * b_per_w
        pltpu.sync_copy(idx_hbm.at[pl.ds(base, b_per_w)], idx_v)

        def gather(i, b):
            return pltpu.make_async_copy(
                table_hbm.at[idx_v.at[pl.ds(i * chunk, chunk)]], rows[b], gsem[b]
            )

        def out_copy(i, b):
            return pltpu.make_async_copy(
                rows[b], out_hbm.at[pl.ds(base + i * chunk, chunk)], osem[b]
            )

        for i in range(nbuf):
            gather(i, i).start()
        for i in range(n_chunks):
            b = i % nbuf
            gather(i, b).wait()
            out_copy(i, b).start()
            j = i + nbuf
            if j < n_chunks:
                out_copy(i, b).wait()
                gather(j, b).start()
        for i in range(n_chunks - nbuf, n_chunks):
            out_copy(i, i % nbuf).wait()

    return gather_kernel


def kernel(x, table):
    b, s = x.shape
    idx_flat = x.reshape(b * s).astype(jnp.int32)
    out = _make_gather(b * s, table.shape[0])(table, idx_flat)
    return out.reshape(b, s, _D)
